# baseline (speedup 1.0000x reference)
import functools

import jax
import jax.numpy as jnp
from jax import lax
from jax.experimental import pallas as pl
from jax.experimental.pallas import tpu as pltpu
from jax.experimental.pallas import tpu_sc as plsc

N = 10000
E = 320000
D = 128
NC = 2
NS = 16
NW = NC * NS
CH = 128
NCHUNKS = E // CH
CBASE = NCHUNKS // NW
EXTRA = NCHUNKS - CBASE * NW
N_PAD = 10240
RPT = N_PAD // NS
ZCH = 16
WCH = 128
DPT = N_PAD // NS

_mesh = plsc.VectorSubcoreMesh(core_axis_name="c", subcore_axis_name="s")


def _zero_block(buf, rows):
    def body(i, _):
        for j in range(D // 16):
            buf[i, pl.ds(j * 16, 16)] = jnp.zeros((16,), jnp.float32)
        return 0
    lax.fori_loop(0, rows, body, 0)


DNIS = 4


@functools.partial(
    pl.kernel,
    out_type=jax.ShapeDtypeStruct((NC, N_PAD), jnp.float32),
    mesh=_mesh,
    scratch_types=[
        pltpu.VMEM((DNIS, 1, 128), jnp.int32),
        pltpu.VMEM((128,), jnp.float32),
        pltpu.VMEM((DPT,), jnp.float32),
        pltpu.VMEM_SHARED((N_PAD,), jnp.float32),
        pltpu.SemaphoreType.DMA((DNIS,)),
        pltpu.SemaphoreType.DMA((2,)),
        pltpu.SemaphoreType.DMA,
    ],
)
def _sc_degree(dst_hbm, out_hbm, dst_v, ones_v, zbuf_v, acc_sh,
               idsem, ssem, zsem):
    c = lax.axis_index("c")
    s = lax.axis_index("s")
    w = s * NC + c
    nch = CBASE + jnp.where(w < EXTRA, 1, 0)
    base = w * CBASE + jnp.minimum(w, EXTRA)

    for j in range(DPT // 16):
        zbuf_v[pl.ds(j * 16, 16)] = jnp.zeros((16,), jnp.float32)
    for j in range(128 // 16):
        ones_v[pl.ds(j * 16, 16)] = jnp.ones((16,), jnp.float32)
    zd = pltpu.async_copy(zbuf_v, acc_sh.at[pl.ds(s * DPT, DPT)], zsem)

    for j in (0, 1):
        pltpu.async_copy(dst_hbm.at[base + j], dst_v.at[j], idsem.at[j])
    zd.wait()
    plsc.subcore_barrier()

    def body(j, _):
        isl = j % DNIS
        sc = j % 2
        pltpu.make_async_copy(dst_hbm.at[0], dst_v.at[isl],
                              idsem.at[isl]).wait()

        @pl.when(j >= 2)
        def _():
            pltpu.make_async_copy(ones_v, acc_sh.at[dst_v.at[0, 0]],
                                  ssem.at[sc]).wait()
        pltpu.async_copy(ones_v, acc_sh.at[dst_v.at[isl, 0]],
                         ssem.at[sc], add=True)

        @pl.when(j + 2 < nch)
        def _():
            pisl = (j + 2) % DNIS
            pltpu.async_copy(dst_hbm.at[base + j + 2], dst_v.at[pisl],
                             idsem.at[pisl])
        return 0
    lax.fori_loop(0, nch, body, 0)

    for sc in range(2):
        pltpu.make_async_copy(ones_v, acc_sh.at[dst_v.at[0, 0]],
                              ssem.at[sc]).wait()
    plsc.subcore_barrier()

    pltpu.sync_copy(acc_sh.at[pl.ds(s * DPT, DPT)],
                    out_hbm.at[c, pl.ds(s * DPT, DPT)])


NRS = 2
NIS = 4


@functools.partial(
    pl.kernel,
    out_type=jax.ShapeDtypeStruct((NC, N_PAD, D), jnp.float32),
    mesh=_mesh,
    scratch_types=[
        pltpu.VMEM((NIS, 1, CH), jnp.int32),
        pltpu.VMEM((NIS, 1, CH), jnp.int32),
        pltpu.VMEM((NRS, CH, D), jnp.float32),
        pltpu.VMEM((ZCH, D), jnp.float32),
        pltpu.VMEM_SHARED((N_PAD, D), jnp.float32),
        pltpu.SemaphoreType.DMA((NRS,)),
        pltpu.SemaphoreType.DMA((NRS,)),
        pltpu.SemaphoreType.DMA((NIS,)),
        pltpu.SemaphoreType.DMA((NIS,)),
        pltpu.SemaphoreType.DMA,
    ],
)
def _sc_segsum(x_hbm, src_hbm, dst_hbm, out_hbm,
               src_v, dst_v, rows_v, zbuf_v, acc_sh,
               gsem, ssem, issem, idsem, zwsem):
    c = lax.axis_index("c")
    s = lax.axis_index("s")
    w = s * NC + c
    nch = CBASE + jnp.where(w < EXTRA, 1, 0)
    base = w * CBASE + jnp.minimum(w, EXTRA)

    _zero_block(zbuf_v, ZCH)
    zdescs = [
        pltpu.async_copy(zbuf_v, acc_sh.at[pl.ds(s * RPT + k * ZCH, ZCH)],
                         zwsem)
        for k in range(RPT // ZCH)
    ]

    for j in (0, 1):
        pltpu.async_copy(src_hbm.at[base + j], src_v.at[j], issem.at[j])
        pltpu.async_copy(dst_hbm.at[base + j], dst_v.at[j], idsem.at[j])
    pltpu.make_async_copy(src_hbm.at[0], src_v.at[0], issem.at[0]).wait()
    pltpu.make_async_copy(dst_hbm.at[0], dst_v.at[0], idsem.at[0]).wait()
    pltpu.async_copy(x_hbm.at[src_v.at[0, 0]], rows_v.at[0], gsem.at[0])

    for d in zdescs:
        d.wait()
    plsc.subcore_barrier()

    def body(j, _):
        rs = j % NRS
        nrs = (j + 1) % NRS
        isl = j % NIS
        pltpu.make_async_copy(x_hbm.at[pl.ds(0, CH)], rows_v.at[rs],
                              gsem.at[rs]).wait()
        pltpu.async_copy(rows_v.at[rs], acc_sh.at[dst_v.at[isl, 0]],
                         ssem.at[rs], add=True)

        @pl.when(j + 1 < nch)
        def _():
            nisl = (j + 1) % NIS

            @pl.when(j >= 1)
            def _():
                pltpu.make_async_copy(x_hbm.at[pl.ds(0, CH)],
                                      rows_v.at[nrs], ssem.at[nrs]).wait()
            pltpu.make_async_copy(src_hbm.at[0], src_v.at[nisl],
                                  issem.at[nisl]).wait()
            pltpu.make_async_copy(dst_hbm.at[0], dst_v.at[nisl],
                                  idsem.at[nisl]).wait()
            pltpu.async_copy(x_hbm.at[src_v.at[nisl, 0]], rows_v.at[nrs],
                             gsem.at[nrs])

        @pl.when(j + 2 < nch)
        def _():
            pisl = (j + 2) % NIS
            pltpu.async_copy(src_hbm.at[base + j + 2], src_v.at[pisl],
                             issem.at[pisl])
            pltpu.async_copy(dst_hbm.at[base + j + 2], dst_v.at[pisl],
                             idsem.at[pisl])
        return 0
    lax.fori_loop(0, nch, body, 0)

    for rs in range(NRS):
        pltpu.make_async_copy(x_hbm.at[pl.ds(0, CH)], rows_v.at[rs],
                              ssem.at[rs]).wait()
    plsc.subcore_barrier()

    wdescs = [
        pltpu.async_copy(acc_sh.at[pl.ds(s * RPT + k * WCH, WCH)],
                         out_hbm.at[c, pl.ds(s * RPT + k * WCH, WCH)],
                         zwsem)
        for k in range(RPT // WCH)
    ]
    for d in wdescs:
        d.wait()


_RB = 2000


def _prep_body(d0_ref, d1_ref, h_ref, dinv_ref, xs_ref):
    deg = d0_ref[...] + d1_ref[...]
    dinv = lax.rsqrt(jnp.maximum(deg, 1.0))
    dinv_ref[...] = dinv
    xs_ref[...] = h_ref[...] * dinv


def _tc_prep(d0, d1, h):
    grid = (N // _RB,)
    col = pl.BlockSpec((_RB, 1), lambda i: (i, 0))
    mat = pl.BlockSpec((_RB, D), lambda i: (i, 0))
    return pl.pallas_call(
        _prep_body,
        grid=grid,
        in_specs=[col, col, mat],
        out_specs=[col, mat],
        out_shape=[jax.ShapeDtypeStruct((N, 1), jnp.float32),
                   jax.ShapeDtypeStruct((N, D), jnp.float32)],
    )(d0, d1, h)


def _mid_body(a0_ref, a1_ref, dinv_ref, x1_ref, xs1_ref):
    dinv = dinv_ref[...]
    x1 = -(a0_ref[...] + a1_ref[...]) * dinv
    x1_ref[...] = x1
    xs1_ref[...] = x1 * dinv


def _tc_mid(a0, a1, dinv):
    grid = (N // _RB,)
    col = pl.BlockSpec((_RB, 1), lambda i: (i, 0))
    mat = pl.BlockSpec((_RB, D), lambda i: (i, 0))
    return pl.pallas_call(
        _mid_body,
        grid=grid,
        in_specs=[mat, mat, col],
        out_specs=[mat, mat],
        out_shape=[jax.ShapeDtypeStruct((N, D), jnp.float32),
                   jax.ShapeDtypeStruct((N, D), jnp.float32)],
    )(a0, a1, dinv)


def _layer_body(x0_ref, x1_ref, a0_ref, a1_ref, dinv_ref, w_ref, b_ref,
                h_ref, xsn_ref):
    dinv = dinv_ref[...]
    x0 = x0_ref[...]
    x1 = x1_ref[...]
    x2 = -2.0 * (a0_ref[...] + a1_ref[...]) * dinv - x0
    z = jnp.dot(x0, w_ref[0:D, :], preferred_element_type=jnp.float32)
    z += jnp.dot(x1, w_ref[D:2 * D, :], preferred_element_type=jnp.float32)
    z += jnp.dot(x2, w_ref[2 * D:3 * D, :], preferred_element_type=jnp.float32)
    h = jnp.maximum(z + b_ref[...], 0.0)
    h_ref[...] = h
    xsn_ref[...] = h * dinv


def _tc_layer(x0, x1, a0, a1, dinv, w, b):
    grid = (N // _RB,)
    col = pl.BlockSpec((_RB, 1), lambda i: (i, 0))
    mat = pl.BlockSpec((_RB, D), lambda i: (i, 0))
    wspec = pl.BlockSpec((3 * D, D), lambda i: (0, 0))
    bspec = pl.BlockSpec((1, D), lambda i: (0, 0))
    return pl.pallas_call(
        _layer_body,
        grid=grid,
        in_specs=[mat, mat, mat, mat, col, wspec, bspec],
        out_specs=[mat, mat],
        out_shape=[jax.ShapeDtypeStruct((N, D), jnp.float32),
                   jax.ShapeDtypeStruct((N, D), jnp.float32)],
    )(x0, x1, a0, a1, dinv, w, b)


def _readout_body(h_ref, w0_ref, b0_ref, w1_ref, b1_ref, w2_ref, b2_ref,
                  y_ref):
    hg = jnp.sum(h_ref[...], axis=0, keepdims=True) * (1.0 / N)
    y = jnp.maximum(
        jnp.dot(hg, w0_ref[...], preferred_element_type=jnp.float32)
        + b0_ref[...], 0.0)
    y = jnp.maximum(
        jnp.dot(y, w1_ref[...], preferred_element_type=jnp.float32)
        + b1_ref[...], 0.0)
    y_ref[...] = (jnp.dot(y, w2_ref[...], preferred_element_type=jnp.float32)
                  + b2_ref[...])


def _tc_readout(h, w0, b0, w1, b1, w2, b2):
    full = lambda a: pl.BlockSpec(a.shape, lambda: tuple(0 for _ in a.shape))
    return pl.pallas_call(
        _readout_body,
        in_specs=[full(h), full(w0), full(b0), full(w1), full(b1),
                  full(w2), full(b2)],
        out_specs=pl.BlockSpec((1, 3), lambda: (0, 0)),
        out_shape=jax.ShapeDtypeStruct((1, 3), jnp.float32),
    )(h, w0, b0, w1, b1, w2, b2)


def kernel(h, edge_index, W0, b0, W1, b1, W2, b2,
           Wr0, br0, Wr1, br1, Wr2, br2):
    src = edge_index[0].reshape(NCHUNKS, 1, CH)
    dst = edge_index[1].reshape(NCHUNKS, 1, CH)

    degp = _sc_degree(dst)
    d0 = degp[0, :N, None]
    d1 = degp[1, :N, None]
    dinv, xs = _tc_prep(d0, d1, h)

    x = h
    for W, b in ((W0, b0), (W1, b1), (W2, b2)):
        a1 = _sc_segsum(xs, src, dst)
        x1, xs1 = _tc_mid(a1[0, :N], a1[1, :N], dinv)
        a2 = _sc_segsum(xs1, src, dst)
        x, xs = _tc_layer(x, x1, a2[0, :N], a2[1, :N], dinv, W,
                          b.reshape(1, D))

    return _tc_readout(x, Wr0, br0.reshape(1, -1), Wr1, br1.reshape(1, -1),
                       Wr2, br2.reshape(1, -1))


# speedup vs baseline: 1.0066x; 1.0066x over previous
import functools

import jax
import jax.numpy as jnp
from jax import lax
from jax.experimental import pallas as pl
from jax.experimental.pallas import tpu as pltpu
from jax.experimental.pallas import tpu_sc as plsc

N = 10000
E = 320000
D = 128
NC = 2
NS = 16
NW = NC * NS
CH = 80
NCHUNKS = E // CH
CPW = NCHUNKS // NW
DCH = 128
DNCHUNKS = E // DCH
DCB = DNCHUNKS // NW
DEX = DNCHUNKS - DCB * NW
N_PAD = 10240
RPT = N_PAD // NS
ZCH = 16
WCH = 128
DPT = N_PAD // NS

_mesh = plsc.VectorSubcoreMesh(core_axis_name="c", subcore_axis_name="s")


def _zero_block(buf, rows):
    def body(i, _):
        for j in range(D // 16):
            buf[i, pl.ds(j * 16, 16)] = jnp.zeros((16,), jnp.float32)
        return 0
    lax.fori_loop(0, rows, body, 0)


DNIS = 4


@functools.partial(
    pl.kernel,
    out_type=jax.ShapeDtypeStruct((NC, N_PAD), jnp.float32),
    mesh=_mesh,
    scratch_types=[
        pltpu.VMEM((DNIS, 1, 128), jnp.int32),
        pltpu.VMEM((128,), jnp.float32),
        pltpu.VMEM((DPT,), jnp.float32),
        pltpu.VMEM_SHARED((N_PAD,), jnp.float32),
        pltpu.SemaphoreType.DMA((DNIS,)),
        pltpu.SemaphoreType.DMA((2,)),
        pltpu.SemaphoreType.DMA,
    ],
)
def _sc_degree(dst_hbm, out_hbm, dst_v, ones_v, zbuf_v, acc_sh,
               idsem, ssem, zsem):
    c = lax.axis_index("c")
    s = lax.axis_index("s")
    w = s * NC + c
    nch = DCB + jnp.where(w < DEX, 1, 0)
    base = w * DCB + jnp.minimum(w, DEX)

    for j in range(DPT // 16):
        zbuf_v[pl.ds(j * 16, 16)] = jnp.zeros((16,), jnp.float32)
    for j in range(128 // 16):
        ones_v[pl.ds(j * 16, 16)] = jnp.ones((16,), jnp.float32)
    zd = pltpu.async_copy(zbuf_v, acc_sh.at[pl.ds(s * DPT, DPT)], zsem)

    for j in (0, 1):
        pltpu.async_copy(dst_hbm.at[base + j], dst_v.at[j], idsem.at[j])
    zd.wait()
    plsc.subcore_barrier()

    def body(j, _):
        isl = j % DNIS
        sc = j % 2
        pltpu.make_async_copy(dst_hbm.at[0], dst_v.at[isl],
                              idsem.at[isl]).wait()

        @pl.when(j >= 2)
        def _():
            pltpu.make_async_copy(ones_v, acc_sh.at[dst_v.at[0, 0]],
                                  ssem.at[sc]).wait()
        pltpu.async_copy(ones_v, acc_sh.at[dst_v.at[isl, 0]],
                         ssem.at[sc], add=True)

        @pl.when(j + 2 < nch)
        def _():
            pisl = (j + 2) % DNIS
            pltpu.async_copy(dst_hbm.at[base + j + 2], dst_v.at[pisl],
                             idsem.at[pisl])
        return 0
    lax.fori_loop(0, nch, body, 0)

    for sc in range(2):
        pltpu.make_async_copy(ones_v, acc_sh.at[dst_v.at[0, 0]],
                              ssem.at[sc]).wait()
    plsc.subcore_barrier()

    pltpu.sync_copy(acc_sh.at[pl.ds(s * DPT, DPT)],
                    out_hbm.at[c, pl.ds(s * DPT, DPT)])


NRS = 3
NIS = 4


@functools.partial(
    pl.kernel,
    out_type=jax.ShapeDtypeStruct((NC, N_PAD, D), jnp.float32),
    mesh=_mesh,
    scratch_types=[
        pltpu.VMEM((NIS, 2, CH), jnp.int32),
        pltpu.VMEM((NRS, CH, D), jnp.float32),
        pltpu.VMEM((ZCH, D), jnp.float32),
        pltpu.VMEM_SHARED((N_PAD, D), jnp.float32),
        pltpu.SemaphoreType.DMA((NRS,)),
        pltpu.SemaphoreType.DMA((NRS,)),
        pltpu.SemaphoreType.DMA((NIS,)),
        pltpu.SemaphoreType.DMA,
    ],
)
def _sc_segsum(x_hbm, idx_hbm, out_hbm,
               idx_v, rows_v, zbuf_v, acc_sh, gsem, ssem, isem, zwsem):
    c = lax.axis_index("c")
    s = lax.axis_index("s")
    w = s * NC + c
    base = w * CPW

    _zero_block(zbuf_v, ZCH)
    zdescs = [
        pltpu.async_copy(zbuf_v, acc_sh.at[pl.ds(s * RPT + k * ZCH, ZCH)],
                         zwsem)
        for k in range(RPT // ZCH)
    ]

    for j in (0, 1):
        pltpu.async_copy(idx_hbm.at[base + j], idx_v.at[j], isem.at[j])
    pltpu.make_async_copy(idx_hbm.at[0], idx_v.at[0], isem.at[0]).wait()
    pltpu.async_copy(x_hbm.at[idx_v.at[0, 0]], rows_v.at[0], gsem.at[0])

    for d in zdescs:
        d.wait()
    plsc.subcore_barrier()

    def body(j, _):
        rs = j % NRS
        nrs = (j + 1) % NRS
        isl = j % NIS

        @pl.when(j + 1 < CPW)
        def _():
            nisl = (j + 1) % NIS

            @pl.when(j >= NRS - 1)
            def _():
                pltpu.make_async_copy(x_hbm.at[pl.ds(0, CH)],
                                      rows_v.at[nrs], ssem.at[nrs]).wait()
            pltpu.make_async_copy(idx_hbm.at[0], idx_v.at[nisl],
                                  isem.at[nisl]).wait()
            pltpu.async_copy(x_hbm.at[idx_v.at[nisl, 0]], rows_v.at[nrs],
                             gsem.at[nrs])

        pltpu.make_async_copy(x_hbm.at[pl.ds(0, CH)], rows_v.at[rs],
                              gsem.at[rs]).wait()
        pltpu.async_copy(rows_v.at[rs], acc_sh.at[idx_v.at[isl, 1]],
                         ssem.at[rs], add=True)

        @pl.when(j + 2 < CPW)
        def _():
            pisl = (j + 2) % NIS
            pltpu.async_copy(idx_hbm.at[base + j + 2], idx_v.at[pisl],
                             isem.at[pisl])
        return 0
    lax.fori_loop(0, CPW, body, 0)

    for j in range(CPW - NRS, CPW):
        pltpu.make_async_copy(x_hbm.at[pl.ds(0, CH)], rows_v.at[j % NRS],
                              ssem.at[j % NRS]).wait()
    plsc.subcore_barrier()

    wdescs = [
        pltpu.async_copy(acc_sh.at[pl.ds(s * RPT + k * WCH, WCH)],
                         out_hbm.at[c, pl.ds(s * RPT + k * WCH, WCH)],
                         zwsem)
        for k in range(RPT // WCH)
    ]
    for d in wdescs:
        d.wait()


_RB = 2000


def _prep_body(d0_ref, d1_ref, h_ref, dinv_ref, xs_ref):
    deg = d0_ref[...] + d1_ref[...]
    dinv = lax.rsqrt(jnp.maximum(deg, 1.0))
    dinv_ref[...] = dinv
    xs_ref[...] = h_ref[...] * dinv


def _tc_prep(d0, d1, h):
    grid = (N // _RB,)
    col = pl.BlockSpec((_RB, 1), lambda i: (i, 0))
    mat = pl.BlockSpec((_RB, D), lambda i: (i, 0))
    return pl.pallas_call(
        _prep_body,
        grid=grid,
        in_specs=[col, col, mat],
        out_specs=[col, mat],
        out_shape=[jax.ShapeDtypeStruct((N, 1), jnp.float32),
                   jax.ShapeDtypeStruct((N, D), jnp.float32)],
    )(d0, d1, h)


def _mid_body(a0_ref, a1_ref, dinv_ref, x1_ref, xs1_ref):
    dinv = dinv_ref[...]
    x1 = -(a0_ref[...] + a1_ref[...]) * dinv
    x1_ref[...] = x1
    xs1_ref[...] = x1 * dinv


def _tc_mid(a0, a1, dinv):
    grid = (N // _RB,)
    col = pl.BlockSpec((_RB, 1), lambda i: (i, 0))
    mat = pl.BlockSpec((_RB, D), lambda i: (i, 0))
    return pl.pallas_call(
        _mid_body,
        grid=grid,
        in_specs=[mat, mat, col],
        out_specs=[mat, mat],
        out_shape=[jax.ShapeDtypeStruct((N, D), jnp.float32),
                   jax.ShapeDtypeStruct((N, D), jnp.float32)],
    )(a0, a1, dinv)


def _layer_body(x0_ref, x1_ref, a0_ref, a1_ref, dinv_ref, w_ref, b_ref,
                h_ref, xsn_ref):
    dinv = dinv_ref[...]
    x0 = x0_ref[...]
    x1 = x1_ref[...]
    x2 = -2.0 * (a0_ref[...] + a1_ref[...]) * dinv - x0
    z = jnp.dot(x0, w_ref[0:D, :], preferred_element_type=jnp.float32)
    z += jnp.dot(x1, w_ref[D:2 * D, :], preferred_element_type=jnp.float32)
    z += jnp.dot(x2, w_ref[2 * D:3 * D, :], preferred_element_type=jnp.float32)
    h = jnp.maximum(z + b_ref[...], 0.0)
    h_ref[...] = h
    xsn_ref[...] = h * dinv


def _tc_layer(x0, x1, a0, a1, dinv, w, b):
    grid = (N // _RB,)
    col = pl.BlockSpec((_RB, 1), lambda i: (i, 0))
    mat = pl.BlockSpec((_RB, D), lambda i: (i, 0))
    wspec = pl.BlockSpec((3 * D, D), lambda i: (0, 0))
    bspec = pl.BlockSpec((1, D), lambda i: (0, 0))
    return pl.pallas_call(
        _layer_body,
        grid=grid,
        in_specs=[mat, mat, mat, mat, col, wspec, bspec],
        out_specs=[mat, mat],
        out_shape=[jax.ShapeDtypeStruct((N, D), jnp.float32),
                   jax.ShapeDtypeStruct((N, D), jnp.float32)],
    )(x0, x1, a0, a1, dinv, w, b)


def _readout_body(h_ref, w0_ref, b0_ref, w1_ref, b1_ref, w2_ref, b2_ref,
                  y_ref):
    hg = jnp.sum(h_ref[...], axis=0, keepdims=True) * (1.0 / N)
    y = jnp.maximum(
        jnp.dot(hg, w0_ref[...], preferred_element_type=jnp.float32)
        + b0_ref[...], 0.0)
    y = jnp.maximum(
        jnp.dot(y, w1_ref[...], preferred_element_type=jnp.float32)
        + b1_ref[...], 0.0)
    y_ref[...] = (jnp.dot(y, w2_ref[...], preferred_element_type=jnp.float32)
                  + b2_ref[...])


def _tc_readout(h, w0, b0, w1, b1, w2, b2):
    full = lambda a: pl.BlockSpec(a.shape, lambda: tuple(0 for _ in a.shape))
    return pl.pallas_call(
        _readout_body,
        in_specs=[full(h), full(w0), full(b0), full(w1), full(b1),
                  full(w2), full(b2)],
        out_specs=pl.BlockSpec((1, 3), lambda: (0, 0)),
        out_shape=jax.ShapeDtypeStruct((1, 3), jnp.float32),
    )(h, w0, b0, w1, b1, w2, b2)


def kernel(h, edge_index, W0, b0, W1, b1, W2, b2,
           Wr0, br0, Wr1, br1, Wr2, br2):
    idx = jnp.stack([edge_index[0].reshape(NCHUNKS, CH),
                     edge_index[1].reshape(NCHUNKS, CH)], axis=1)

    degp = _sc_degree(edge_index[1].reshape(DNCHUNKS, 1, DCH))
    d0 = degp[0, :N, None]
    d1 = degp[1, :N, None]
    dinv, xs = _tc_prep(d0, d1, h)

    x = h
    for W, b in ((W0, b0), (W1, b1), (W2, b2)):
        a1 = _sc_segsum(xs, idx)
        x1, xs1 = _tc_mid(a1[0, :N], a1[1, :N], dinv)
        a2 = _sc_segsum(xs1, idx)
        x, xs = _tc_layer(x, x1, a2[0, :N], a2[1, :N], dinv, W,
                          b.reshape(1, D))

    return _tc_readout(x, Wr0, br0.reshape(1, -1), Wr1, br1.reshape(1, -1),
                       Wr2, br2.reshape(1, -1))


# speedup vs baseline: 1.1715x; 1.1638x over previous
import functools

import jax
import jax.numpy as jnp
from jax import lax
from jax.experimental import pallas as pl
from jax.experimental.pallas import tpu as pltpu
from jax.experimental.pallas import tpu_sc as plsc

N = 10000
E = 320000
D = 128
NC = 2
NS = 16
NW = NC * NS
CH = 128
NCHUNKS = E // CH
CPB = NCHUNKS // NW
CPX = NCHUNKS - CPB * NW
DCH = 128
DNCHUNKS = E // DCH
DCB = DNCHUNKS // NW
DEX = DNCHUNKS - DCB * NW
N_PAD = 10240
RPT = N_PAD // NS
ZCH = 16
WCH = 128
DPT = N_PAD // NS

_mesh = plsc.VectorSubcoreMesh(core_axis_name="c", subcore_axis_name="s")


def _zero_block(buf, rows):
    def body(i, _):
        for j in range(D // 16):
            buf[i, pl.ds(j * 16, 16)] = jnp.zeros((16,), jnp.float32)
        return 0
    lax.fori_loop(0, rows, body, 0)


DNIS = 4


@functools.partial(
    pl.kernel,
    out_type=jax.ShapeDtypeStruct((NC, N_PAD), jnp.float32),
    mesh=_mesh,
    scratch_types=[
        pltpu.VMEM((DNIS, 1, 128), jnp.int32),
        pltpu.VMEM((128,), jnp.float32),
        pltpu.VMEM((DPT,), jnp.float32),
        pltpu.VMEM_SHARED((N_PAD,), jnp.float32),
        pltpu.SemaphoreType.DMA((DNIS,)),
        pltpu.SemaphoreType.DMA((2,)),
        pltpu.SemaphoreType.DMA,
    ],
)
def _sc_degree(dst_hbm, out_hbm, dst_v, ones_v, zbuf_v, acc_sh,
               idsem, ssem, zsem):
    c = lax.axis_index("c")
    s = lax.axis_index("s")
    w = s * NC + c
    nch = DCB + jnp.where(w < DEX, 1, 0)
    base = w * DCB + jnp.minimum(w, DEX)

    for j in range(DPT // 16):
        zbuf_v[pl.ds(j * 16, 16)] = jnp.zeros((16,), jnp.float32)
    for j in range(128 // 16):
        ones_v[pl.ds(j * 16, 16)] = jnp.ones((16,), jnp.float32)
    zd = pltpu.async_copy(zbuf_v, acc_sh.at[pl.ds(s * DPT, DPT)], zsem)

    for j in (0, 1):
        pltpu.async_copy(dst_hbm.at[base + j], dst_v.at[j], idsem.at[j])
    zd.wait()
    plsc.subcore_barrier()

    def body(j, _):
        isl = j % DNIS
        sc = j % 2
        pltpu.make_async_copy(dst_hbm.at[0], dst_v.at[isl],
                              idsem.at[isl]).wait()

        @pl.when(j >= 2)
        def _():
            pltpu.make_async_copy(ones_v, acc_sh.at[dst_v.at[0, 0]],
                                  ssem.at[sc]).wait()
        pltpu.async_copy(ones_v, acc_sh.at[dst_v.at[isl, 0]],
                         ssem.at[sc], add=True)

        @pl.when(j + 2 < nch)
        def _():
            pisl = (j + 2) % DNIS
            pltpu.async_copy(dst_hbm.at[base + j + 2], dst_v.at[pisl],
                             idsem.at[pisl])
        return 0
    lax.fori_loop(0, nch, body, 0)

    for sc in range(2):
        pltpu.make_async_copy(ones_v, acc_sh.at[dst_v.at[0, 0]],
                              ssem.at[sc]).wait()
    plsc.subcore_barrier()

    pltpu.sync_copy(acc_sh.at[pl.ds(s * DPT, DPT)],
                    out_hbm.at[c, pl.ds(s * DPT, DPT)])


NRS = 2
NIS = 4


@functools.partial(
    pl.kernel,
    out_type=jax.ShapeDtypeStruct((NC, N_PAD, D), jnp.float32),
    mesh=_mesh,
    scratch_types=[
        pltpu.VMEM((NIS, 2, CH), jnp.int32),
        pltpu.VMEM((NRS, CH, D), jnp.float32),
        pltpu.VMEM((ZCH, D), jnp.float32),
        pltpu.VMEM_SHARED((N_PAD, D), jnp.float32),
        pltpu.SemaphoreType.DMA((NRS,)),
        pltpu.SemaphoreType.DMA((NRS,)),
        pltpu.SemaphoreType.DMA((NIS,)),
        pltpu.SemaphoreType.DMA,
    ],
)
def _sc_segsum(x_hbm, idx_hbm, out_hbm,
               idx_v, rows_v, zbuf_v, acc_sh, gsem, ssem, isem, zwsem):
    c = lax.axis_index("c")
    s = lax.axis_index("s")
    w = s * NC + c
    nch = CPB + jnp.where(w < CPX, 1, 0)
    base = w * CPB + jnp.minimum(w, CPX)

    _zero_block(zbuf_v, ZCH)
    zdescs = [
        pltpu.async_copy(zbuf_v, acc_sh.at[pl.ds(s * RPT + k * ZCH, ZCH)],
                         zwsem)
        for k in range(RPT // ZCH)
    ]

    for j in (0, 1):
        pltpu.async_copy(idx_hbm.at[base + j], idx_v.at[j], isem.at[j])
    pltpu.make_async_copy(idx_hbm.at[0], idx_v.at[0], isem.at[0]).wait()
    pltpu.async_copy(x_hbm.at[idx_v.at[0, 0]], rows_v.at[0], gsem.at[0])

    for d in zdescs:
        d.wait()
    plsc.subcore_barrier()

    def body(j, _):
        rs = j % NRS
        nrs = (j + 1) % NRS
        isl = j % NIS

        @pl.when(j + 1 < nch)
        def _():
            nisl = (j + 1) % NIS

            @pl.when(j >= NRS - 1)
            def _():
                pltpu.make_async_copy(x_hbm.at[pl.ds(0, CH)],
                                      rows_v.at[nrs], ssem.at[nrs]).wait()
            pltpu.make_async_copy(idx_hbm.at[0], idx_v.at[nisl],
                                  isem.at[nisl]).wait()
            pltpu.async_copy(x_hbm.at[idx_v.at[nisl, 0]], rows_v.at[nrs],
                             gsem.at[nrs])

        pltpu.make_async_copy(x_hbm.at[pl.ds(0, CH)], rows_v.at[rs],
                              gsem.at[rs]).wait()
        pltpu.async_copy(rows_v.at[rs], acc_sh.at[idx_v.at[isl, 1]],
                         ssem.at[rs], add=True)

        @pl.when(j + 2 < nch)
        def _():
            pisl = (j + 2) % NIS
            pltpu.async_copy(idx_hbm.at[base + j + 2], idx_v.at[pisl],
                             isem.at[pisl])
        return 0
    lax.fori_loop(0, nch, body, 0)

    for rs in range(NRS):
        pltpu.make_async_copy(x_hbm.at[pl.ds(0, CH)], rows_v.at[rs],
                              ssem.at[rs]).wait()
    plsc.subcore_barrier()

    wdescs = [
        pltpu.async_copy(acc_sh.at[pl.ds(s * RPT + k * WCH, WCH)],
                         out_hbm.at[c, pl.ds(s * RPT + k * WCH, WCH)],
                         zwsem)
        for k in range(RPT // WCH)
    ]
    for d in wdescs:
        d.wait()


_RB = 2000


def _prep_body(d0_ref, d1_ref, h_ref, dinv_ref, xs_ref):
    deg = d0_ref[...] + d1_ref[...]
    dinv = lax.rsqrt(jnp.maximum(deg, 1.0))
    dinv_ref[...] = dinv
    xs_ref[...] = h_ref[...] * dinv


def _tc_prep(d0, d1, h):
    grid = (N // _RB,)
    col = pl.BlockSpec((_RB, 1), lambda i: (i, 0))
    mat = pl.BlockSpec((_RB, D), lambda i: (i, 0))
    return pl.pallas_call(
        _prep_body,
        grid=grid,
        in_specs=[col, col, mat],
        out_specs=[col, mat],
        out_shape=[jax.ShapeDtypeStruct((N, 1), jnp.float32),
                   jax.ShapeDtypeStruct((N, D), jnp.float32)],
    )(d0, d1, h)


def _mid_body(a0_ref, a1_ref, dinv_ref, x1_ref, xs1_ref):
    dinv = dinv_ref[...]
    x1 = -(a0_ref[...] + a1_ref[...]) * dinv
    x1_ref[...] = x1
    xs1_ref[...] = x1 * dinv


def _tc_mid(a0, a1, dinv):
    grid = (N // _RB,)
    col = pl.BlockSpec((_RB, 1), lambda i: (i, 0))
    mat = pl.BlockSpec((_RB, D), lambda i: (i, 0))
    return pl.pallas_call(
        _mid_body,
        grid=grid,
        in_specs=[mat, mat, col],
        out_specs=[mat, mat],
        out_shape=[jax.ShapeDtypeStruct((N, D), jnp.float32),
                   jax.ShapeDtypeStruct((N, D), jnp.float32)],
    )(a0, a1, dinv)


def _layer_body(x0_ref, x1_ref, a0_ref, a1_ref, dinv_ref, w_ref, b_ref,
                h_ref, xsn_ref):
    dinv = dinv_ref[...]
    x0 = x0_ref[...]
    x1 = x1_ref[...]
    x2 = -2.0 * (a0_ref[...] + a1_ref[...]) * dinv - x0
    z = jnp.dot(x0, w_ref[0:D, :], preferred_element_type=jnp.float32)
    z += jnp.dot(x1, w_ref[D:2 * D, :], preferred_element_type=jnp.float32)
    z += jnp.dot(x2, w_ref[2 * D:3 * D, :], preferred_element_type=jnp.float32)
    h = jnp.maximum(z + b_ref[...], 0.0)
    h_ref[...] = h
    xsn_ref[...] = h * dinv


def _tc_layer(x0, x1, a0, a1, dinv, w, b):
    grid = (N // _RB,)
    col = pl.BlockSpec((_RB, 1), lambda i: (i, 0))
    mat = pl.BlockSpec((_RB, D), lambda i: (i, 0))
    wspec = pl.BlockSpec((3 * D, D), lambda i: (0, 0))
    bspec = pl.BlockSpec((1, D), lambda i: (0, 0))
    return pl.pallas_call(
        _layer_body,
        grid=grid,
        in_specs=[mat, mat, mat, mat, col, wspec, bspec],
        out_specs=[mat, mat],
        out_shape=[jax.ShapeDtypeStruct((N, D), jnp.float32),
                   jax.ShapeDtypeStruct((N, D), jnp.float32)],
    )(x0, x1, a0, a1, dinv, w, b)


def _readout_body(h_ref, w0_ref, b0_ref, w1_ref, b1_ref, w2_ref, b2_ref,
                  y_ref):
    hg = jnp.sum(h_ref[...], axis=0, keepdims=True) * (1.0 / N)
    y = jnp.maximum(
        jnp.dot(hg, w0_ref[...], preferred_element_type=jnp.float32)
        + b0_ref[...], 0.0)
    y = jnp.maximum(
        jnp.dot(y, w1_ref[...], preferred_element_type=jnp.float32)
        + b1_ref[...], 0.0)
    y_ref[...] = (jnp.dot(y, w2_ref[...], preferred_element_type=jnp.float32)
                  + b2_ref[...])


def _tc_readout(h, w0, b0, w1, b1, w2, b2):
    full = lambda a: pl.BlockSpec(a.shape, lambda: tuple(0 for _ in a.shape))
    return pl.pallas_call(
        _readout_body,
        in_specs=[full(h), full(w0), full(b0), full(w1), full(b1),
                  full(w2), full(b2)],
        out_specs=pl.BlockSpec((1, 3), lambda: (0, 0)),
        out_shape=jax.ShapeDtypeStruct((1, 3), jnp.float32),
    )(h, w0, b0, w1, b1, w2, b2)


def kernel(h, edge_index, W0, b0, W1, b1, W2, b2,
           Wr0, br0, Wr1, br1, Wr2, br2):
    idx = jnp.stack([edge_index[0].reshape(NCHUNKS, CH),
                     edge_index[1].reshape(NCHUNKS, CH)], axis=1)

    degp = _sc_degree(edge_index[1].reshape(DNCHUNKS, 1, DCH))
    d0 = degp[0, :N, None]
    d1 = degp[1, :N, None]
    dinv, xs = _tc_prep(d0, d1, h)

    x = h
    for W, b in ((W0, b0), (W1, b1), (W2, b2)):
        a1 = _sc_segsum(xs, idx)
        x1, xs1 = _tc_mid(a1[0, :N], a1[1, :N], dinv)
        a2 = _sc_segsum(xs1, idx)
        x, xs = _tc_layer(x, x1, a2[0, :N], a2[1, :N], dinv, W,
                          b.reshape(1, D))

    return _tc_readout(x, Wr0, br0.reshape(1, -1), Wr1, br1.reshape(1, -1),
                       Wr2, br2.reshape(1, -1))
